# Initial kernel scaffold; baseline (speedup 1.0000x reference)
import jax
import jax.numpy as jnp
from jax.experimental import pallas as pl


def kernel(x, mask, W, b):
    raise NotImplementedError("write your pallas kernel here")


# speedup vs baseline: 5.7520x; 5.7520x over previous
import jax
import jax.numpy as jnp
from jax.experimental import pallas as pl
from jax.experimental.pallas import tpu as pltpu

_BLKW = 256
_BLKT = 1024


def _mix_kernel(mask_ref, w_ref, wmix_ref):
    e_dim = w_ref.shape[0]
    acc = w_ref[0] * mask_ref[0]
    for e in range(1, e_dim):
        acc += w_ref[e] * mask_ref[e]
    wmix_ref[...] = acc


def _mm_kernel(mask_ref, x_ref, wmix_ref, b_ref, out_ref):
    e_dim = b_ref.shape[0]
    acc = jnp.dot(x_ref[...], wmix_ref[...],
                  preferred_element_type=jnp.float32)
    bmix = b_ref[0:1, :] * mask_ref[0]
    for e in range(1, e_dim):
        bmix += b_ref[e:e + 1, :] * mask_ref[e]
    out_ref[...] = acc + bmix


def kernel(x, mask, W, b):
    t, d = x.shape
    e = W.shape[0]

    wmix = pl.pallas_call(
        _mix_kernel,
        grid=(d // _BLKW,),
        in_specs=[
            pl.BlockSpec(memory_space=pltpu.MemorySpace.SMEM),
            pl.BlockSpec((e, _BLKW, d), lambda k: (0, k, 0)),
        ],
        out_specs=pl.BlockSpec((_BLKW, d), lambda k: (k, 0)),
        out_shape=jax.ShapeDtypeStruct((d, d), jnp.float32),
    )(mask, W)

    out = pl.pallas_call(
        _mm_kernel,
        grid=(t // _BLKT,),
        in_specs=[
            pl.BlockSpec(memory_space=pltpu.MemorySpace.SMEM),
            pl.BlockSpec((_BLKT, d), lambda i: (i, 0)),
            pl.BlockSpec((d, d), lambda i: (0, 0)),
            pl.BlockSpec((e, d), lambda i: (0, 0)),
        ],
        out_specs=pl.BlockSpec((_BLKT, d), lambda i: (i, 0)),
        out_shape=jax.ShapeDtypeStruct((t, d), jnp.float32),
    )(mask, x, wmix, b)

    return (out, mask)


# speedup vs baseline: 6.3602x; 1.1057x over previous
import jax
import jax.numpy as jnp
from jax.experimental import pallas as pl
from jax.experimental.pallas import tpu as pltpu

_BLKW = 256
_BLKT = 1024


def _fused_kernel(mask_ref, w_ref, x_ref, b_ref, out_ref, wmix_ref):
    s = pl.program_id(0)
    e_dim, blkw, _ = w_ref.shape
    nkw = wmix_ref.shape[0] // blkw

    @pl.when(s < nkw)
    def _mix():
        acc = w_ref[0] * mask_ref[0]
        for e in range(1, e_dim):
            acc += w_ref[e] * mask_ref[e]
        wmix_ref[pl.ds(s * blkw, blkw), :] = acc

    @pl.when(s >= nkw)
    def _matmul():
        acc = jnp.dot(x_ref[...], wmix_ref[...],
                      preferred_element_type=jnp.float32)
        bmix = b_ref[0:1, :] * mask_ref[0]
        for e in range(1, e_dim):
            bmix += b_ref[e:e + 1, :] * mask_ref[e]
        out_ref[...] = acc + bmix


def kernel(x, mask, W, b):
    t, d = x.shape
    e = W.shape[0]
    nkw = d // _BLKW
    nt = t // _BLKT

    out = pl.pallas_call(
        _fused_kernel,
        grid=(nkw + nt,),
        in_specs=[
            pl.BlockSpec(memory_space=pltpu.MemorySpace.SMEM),
            pl.BlockSpec((e, _BLKW, d),
                         lambda s: (0, jnp.minimum(s, nkw - 1), 0)),
            pl.BlockSpec((_BLKT, d),
                         lambda s: (jnp.maximum(s - nkw, 0), 0)),
            pl.BlockSpec((e, d), lambda s: (0, 0)),
        ],
        out_specs=pl.BlockSpec((_BLKT, d),
                               lambda s: (jnp.maximum(s - nkw, 0), 0)),
        out_shape=jax.ShapeDtypeStruct((t, d), jnp.float32),
        scratch_shapes=[pltpu.VMEM((d, d), jnp.float32)],
    )(mask, W, x, b)

    return (out, mask)
